# baseline (speedup 1.0000x reference)
import functools

import jax
import jax.numpy as jnp
from jax import lax
from jax.experimental import pallas as pl
from jax.experimental.pallas import tpu as pltpu
from jax.experimental.pallas import tpu_sc as plsc


T = 128
_CH = 32


@functools.lru_cache(maxsize=None)
def _make_sc_row_gather(R, D, B):
    info = plsc.get_sparse_core_info()
    nw = info.num_cores * info.num_subcores
    b_per_w = B // nw
    assert B % (8 * nw) == 0 and b_per_w % _CH == 0
    n_ch = b_per_w // _CH
    nc = info.num_cores
    mesh = plsc.VectorSubcoreMesh(core_axis_name="c", subcore_axis_name="s")

    @functools.partial(
        pl.kernel,
        out_type=jax.ShapeDtypeStruct((B, D), jnp.float32),
        mesh=mesh,
        scratch_types=[
            pltpu.VMEM((b_per_w,), jnp.int32),
            pltpu.VMEM((_CH, D), jnp.float32),
            pltpu.VMEM((_CH, D), jnp.float32),
            pltpu.SemaphoreType.DMA,
            pltpu.SemaphoreType.DMA,
            pltpu.SemaphoreType.DMA,
        ],
    )
    def k(table_hbm, idx_hbm, out_hbm, idx_v, buf0, buf1, gsem, ssem0, ssem1):
        wid = lax.axis_index("s") * nc + lax.axis_index("c")
        base = wid * b_per_w
        pltpu.sync_copy(idx_hbm.at[pl.ds(base, b_per_w)], idx_v)
        bufs = (buf0, buf1)
        ssems = (ssem0, ssem1)
        for c in range(n_ch):
            buf = bufs[c % 2]
            ssem = ssems[c % 2]
            if c >= 2:
                pltpu.make_async_copy(
                    buf, out_hbm.at[pl.ds(base + (c - 2) * _CH, _CH)], ssem
                ).wait()
            pltpu.async_copy(
                table_hbm.at[idx_v.at[pl.ds(c * _CH, _CH)]], buf, gsem
            ).wait()
            pltpu.async_copy(buf, out_hbm.at[pl.ds(base + c * _CH, _CH)], ssem)
        for c in range(max(n_ch - 2, 0), n_ch):
            pltpu.make_async_copy(
                bufs[c % 2], out_hbm.at[pl.ds(base + c * _CH, _CH)], ssems[c % 2]
            ).wait()

    return k


@functools.lru_cache(maxsize=None)
def _make_sc_row_scatter(B, D, R):
    info = plsc.get_sparse_core_info()
    nw = info.num_cores * info.num_subcores
    b_per_w = B // nw
    assert B % (8 * nw) == 0 and b_per_w % _CH == 0
    n_ch = b_per_w // _CH
    nc = info.num_cores
    mesh = plsc.VectorSubcoreMesh(core_axis_name="c", subcore_axis_name="s")

    @functools.partial(
        pl.kernel,
        out_type=jax.ShapeDtypeStruct((R, D), jnp.float32),
        mesh=mesh,
        scratch_types=[
            pltpu.VMEM((n_ch, _CH), jnp.int32),
            pltpu.VMEM((_CH, D), jnp.float32),
            pltpu.VMEM((_CH, D), jnp.float32),
            pltpu.SemaphoreType.DMA,
            pltpu.SemaphoreType.DMA,
        ],
    )
    def k(table_hbm, idx_hbm, out_hbm, idx_v, buf0, buf1, ssem0, ssem1):
        wid = lax.axis_index("s") * nc + lax.axis_index("c")
        base = wid * b_per_w
        pltpu.sync_copy(idx_hbm.at[wid], idx_v)
        bufs = (buf0, buf1)
        ssems = (ssem0, ssem1)
        for c in range(n_ch):
            buf = bufs[c % 2]
            ssem = ssems[c % 2]
            if c >= 2:
                pltpu.make_async_copy(
                    buf, out_hbm.at[idx_v.at[c - 2]], ssem
                ).wait()
            pltpu.sync_copy(table_hbm.at[pl.ds(base + c * _CH, _CH)], buf)
            pltpu.async_copy(buf, out_hbm.at[idx_v.at[c]], ssem)
        for c in range(max(n_ch - 2, 0), n_ch):
            pltpu.make_async_copy(
                bufs[c % 2], out_hbm.at[idx_v.at[c]], ssems[c % 2]
            ).wait()

    return k


def _mm_body(te_ref, x_ref, w_ref, b_ref, y_ref):
    x = x_ref[...]
    w = w_ref[0]
    y = lax.dot_general(x, w, (((1,), (1,)), ((), ())),
                        preferred_element_type=jnp.float32)
    y_ref[...] = y + b_ref[0]


def _grouped_matmul(tile_expert, x_padded, W, b):
    P, D = x_padded.shape
    nt = P // T
    grid_spec = pltpu.PrefetchScalarGridSpec(
        num_scalar_prefetch=1,
        grid=(nt,),
        in_specs=[
            pl.BlockSpec((T, D), lambda i, te: (i, 0)),
            pl.BlockSpec((1, D, D), lambda i, te: (te[i], 0, 0)),
            pl.BlockSpec((1, 1, D), lambda i, te: (te[i], 0, 0)),
        ],
        out_specs=pl.BlockSpec((T, D), lambda i, te: (i, 0)),
    )
    return pl.pallas_call(
        _mm_body,
        grid_spec=grid_spec,
        out_shape=jax.ShapeDtypeStruct((P, D), jnp.float32),
    )(tile_expert, x_padded, W, b.reshape(b.shape[0], 1, b.shape[1]))


def kernel(xs, mxs, actions, W, b):
    N, D = xs.shape
    E = W.shape[0]
    a = actions.astype(jnp.int32)

    oh = (a[:, None] == jnp.arange(E, dtype=jnp.int32)[None, :]).astype(jnp.int32)
    csum = jnp.cumsum(oh, axis=0)
    hist = csum[-1]
    padded = ((hist + T - 1) // T) * T
    off_end = jnp.cumsum(padded)
    off = off_end - padded
    P = N + E * T
    p = jnp.sum(oh * (off[None, :] + csum - 1), axis=1).astype(jnp.int32)
    tile_start = jnp.arange(P // T, dtype=jnp.int32)[:, None] * T
    tile_expert = jnp.minimum(
        jnp.sum((tile_start >= off_end[None, :]).astype(jnp.int32), axis=1),
        E - 1)

    nw = 32
    p3 = p.reshape(nw, -1, _CH)
    x_padded = _make_sc_row_scatter(N, D, P)(xs, p3)

    y_padded = _grouped_matmul(tile_expert, x_padded, W, b)

    ys = _make_sc_row_gather(P, D, N)(y_padded, p)
    return (ys, mxs, actions)


# speedup vs baseline: 1.1430x; 1.1430x over previous
import functools

import jax
import jax.numpy as jnp
from jax import lax
from jax.experimental import pallas as pl
from jax.experimental.pallas import tpu as pltpu
from jax.experimental.pallas import tpu_sc as plsc


T = 256
_CH = 32


@functools.lru_cache(maxsize=None)
def _make_sc_row_gather(R, D, B):
    info = plsc.get_sparse_core_info()
    nw = info.num_cores * info.num_subcores
    b_per_w = B // nw
    assert B % (8 * nw) == 0 and b_per_w % _CH == 0
    n_ch = b_per_w // _CH
    nc = info.num_cores
    mesh = plsc.VectorSubcoreMesh(core_axis_name="c", subcore_axis_name="s")

    @functools.partial(
        pl.kernel,
        out_type=jax.ShapeDtypeStruct((B, D), jnp.float32),
        mesh=mesh,
        scratch_types=[
            pltpu.VMEM((b_per_w,), jnp.int32),
            pltpu.VMEM((_CH, D), jnp.float32),
            pltpu.VMEM((_CH, D), jnp.float32),
            pltpu.SemaphoreType.DMA,
            pltpu.SemaphoreType.DMA,
            pltpu.SemaphoreType.DMA,
        ],
    )
    def k(table_hbm, idx_hbm, out_hbm, idx_v, buf0, buf1, gsem, ssem0, ssem1):
        wid = lax.axis_index("s") * nc + lax.axis_index("c")
        base = wid * b_per_w
        pltpu.sync_copy(idx_hbm.at[pl.ds(base, b_per_w)], idx_v)
        bufs = (buf0, buf1)
        ssems = (ssem0, ssem1)
        for c in range(n_ch):
            buf = bufs[c % 2]
            ssem = ssems[c % 2]
            if c >= 2:
                pltpu.make_async_copy(
                    buf, out_hbm.at[pl.ds(base + (c - 2) * _CH, _CH)], ssem
                ).wait()
            pltpu.async_copy(
                table_hbm.at[idx_v.at[pl.ds(c * _CH, _CH)]], buf, gsem
            ).wait()
            pltpu.async_copy(buf, out_hbm.at[pl.ds(base + c * _CH, _CH)], ssem)
        for c in range(max(n_ch - 2, 0), n_ch):
            pltpu.make_async_copy(
                bufs[c % 2], out_hbm.at[pl.ds(base + c * _CH, _CH)], ssems[c % 2]
            ).wait()

    return k


@functools.lru_cache(maxsize=None)
def _make_sc_row_scatter(B, D, R):
    info = plsc.get_sparse_core_info()
    nw = info.num_cores * info.num_subcores
    b_per_w = B // nw
    assert B % (8 * nw) == 0 and b_per_w % _CH == 0
    n_ch = b_per_w // _CH
    nc = info.num_cores
    mesh = plsc.VectorSubcoreMesh(core_axis_name="c", subcore_axis_name="s")

    @functools.partial(
        pl.kernel,
        out_type=jax.ShapeDtypeStruct((R, D), jnp.float32),
        mesh=mesh,
        scratch_types=[
            pltpu.VMEM((n_ch, _CH), jnp.int32),
            pltpu.VMEM((_CH, D), jnp.float32),
            pltpu.VMEM((_CH, D), jnp.float32),
            pltpu.SemaphoreType.DMA,
            pltpu.SemaphoreType.DMA,
        ],
    )
    def k(table_hbm, idx_hbm, out_hbm, idx_v, buf0, buf1, ssem0, ssem1):
        wid = lax.axis_index("s") * nc + lax.axis_index("c")
        base = wid * b_per_w
        pltpu.sync_copy(idx_hbm.at[wid], idx_v)
        bufs = (buf0, buf1)
        ssems = (ssem0, ssem1)
        for c in range(n_ch):
            buf = bufs[c % 2]
            ssem = ssems[c % 2]
            if c >= 2:
                pltpu.make_async_copy(
                    buf, out_hbm.at[idx_v.at[c - 2]], ssem
                ).wait()
            pltpu.sync_copy(table_hbm.at[pl.ds(base + c * _CH, _CH)], buf)
            pltpu.async_copy(buf, out_hbm.at[idx_v.at[c]], ssem)
        for c in range(max(n_ch - 2, 0), n_ch):
            pltpu.make_async_copy(
                bufs[c % 2], out_hbm.at[idx_v.at[c]], ssems[c % 2]
            ).wait()

    return k


def _mm_body(te_ref, x_ref, w_ref, b_ref, y_ref):
    x = x_ref[...]
    w = w_ref[0]
    y = lax.dot_general(x, w, (((1,), (1,)), ((), ())),
                        preferred_element_type=jnp.float32)
    y_ref[...] = y + b_ref[0]


def _grouped_matmul(tile_expert, x_padded, W, b):
    P, D = x_padded.shape
    nt = P // T
    grid_spec = pltpu.PrefetchScalarGridSpec(
        num_scalar_prefetch=1,
        grid=(nt,),
        in_specs=[
            pl.BlockSpec((T, D), lambda i, te: (i, 0)),
            pl.BlockSpec((1, D, D), lambda i, te: (te[i], 0, 0)),
            pl.BlockSpec((1, 1, D), lambda i, te: (te[i], 0, 0)),
        ],
        out_specs=pl.BlockSpec((T, D), lambda i, te: (i, 0)),
    )
    return pl.pallas_call(
        _mm_body,
        grid_spec=grid_spec,
        out_shape=jax.ShapeDtypeStruct((P, D), jnp.float32),
    )(tile_expert, x_padded, W, b.reshape(b.shape[0], 1, b.shape[1]))


def kernel(xs, mxs, actions, W, b):
    N, D = xs.shape
    E = W.shape[0]
    a = actions.astype(jnp.int32)

    oh = (a[:, None] == jnp.arange(E, dtype=jnp.int32)[None, :]).astype(jnp.int32)
    csum = jnp.cumsum(oh, axis=0)
    hist = csum[-1]
    padded = ((hist + T - 1) // T) * T
    off_end = jnp.cumsum(padded)
    off = off_end - padded
    P = N + E * T
    p = jnp.sum(oh * (off[None, :] + csum - 1), axis=1).astype(jnp.int32)
    tile_start = jnp.arange(P // T, dtype=jnp.int32)[:, None] * T
    tile_expert = jnp.minimum(
        jnp.sum((tile_start >= off_end[None, :]).astype(jnp.int32), axis=1),
        E - 1)

    nw = 32
    p3 = p.reshape(nw, -1, _CH)
    x_padded = _make_sc_row_scatter(N, D, P)(xs, p3)

    y_padded = _grouped_matmul(tile_expert, x_padded, W, b)

    ys = _make_sc_row_gather(P, D, N)(y_padded, p)
    return (ys, mxs, actions)


# speedup vs baseline: 1.1928x; 1.0435x over previous
import functools

import jax
import jax.numpy as jnp
from jax import lax
from jax.experimental import pallas as pl
from jax.experimental.pallas import tpu as pltpu
from jax.experimental.pallas import tpu_sc as plsc


T = 256
_CH = 32


@functools.lru_cache(maxsize=None)
def _make_sc_row_gather(R, D, B):
    info = plsc.get_sparse_core_info()
    nw = info.num_cores * info.num_subcores
    b_per_w = B // nw
    assert B % (8 * nw) == 0 and b_per_w % _CH == 0
    n_ch = b_per_w // _CH
    nc = info.num_cores
    mesh = plsc.VectorSubcoreMesh(core_axis_name="c", subcore_axis_name="s")

    @functools.partial(
        pl.kernel,
        out_type=jax.ShapeDtypeStruct((B, D), jnp.float32),
        mesh=mesh,
        scratch_types=[
            pltpu.VMEM((b_per_w,), jnp.int32),
            pltpu.VMEM((_CH, D), jnp.float32),
            pltpu.VMEM((_CH, D), jnp.float32),
            pltpu.SemaphoreType.DMA,
            pltpu.SemaphoreType.DMA,
            pltpu.SemaphoreType.DMA,
        ],
    )
    def k(table_hbm, idx_hbm, out_hbm, idx_v, buf0, buf1, gsem, ssem0, ssem1):
        wid = lax.axis_index("s") * nc + lax.axis_index("c")
        base = wid * b_per_w
        pltpu.sync_copy(idx_hbm.at[pl.ds(base, b_per_w)], idx_v)
        bufs = (buf0, buf1)
        ssems = (ssem0, ssem1)
        for c in range(n_ch):
            buf = bufs[c % 2]
            ssem = ssems[c % 2]
            if c >= 2:
                pltpu.make_async_copy(
                    buf, out_hbm.at[pl.ds(base + (c - 2) * _CH, _CH)], ssem
                ).wait()
            pltpu.async_copy(
                table_hbm.at[idx_v.at[pl.ds(c * _CH, _CH)]], buf, gsem
            ).wait()
            pltpu.async_copy(buf, out_hbm.at[pl.ds(base + c * _CH, _CH)], ssem)
        for c in range(max(n_ch - 2, 0), n_ch):
            pltpu.make_async_copy(
                bufs[c % 2], out_hbm.at[pl.ds(base + c * _CH, _CH)], ssems[c % 2]
            ).wait()

    return k


@functools.lru_cache(maxsize=None)
def _make_sc_row_scatter(B, D, R):
    info = plsc.get_sparse_core_info()
    nw = info.num_cores * info.num_subcores
    b_per_w = B // nw
    assert B % (8 * nw) == 0 and b_per_w % _CH == 0
    n_ch = b_per_w // _CH
    nc = info.num_cores
    mesh = plsc.VectorSubcoreMesh(core_axis_name="c", subcore_axis_name="s")

    @functools.partial(
        pl.kernel,
        out_type=jax.ShapeDtypeStruct((R, D), jnp.float32),
        mesh=mesh,
        scratch_types=[
            pltpu.VMEM((n_ch, _CH), jnp.int32),
            pltpu.VMEM((_CH, D), jnp.float32),
            pltpu.VMEM((_CH, D), jnp.float32),
            pltpu.SemaphoreType.DMA,
            pltpu.SemaphoreType.DMA,
        ],
    )
    def k(table_hbm, idx_hbm, out_hbm, idx_v, buf0, buf1, ssem0, ssem1):
        wid = lax.axis_index("s") * nc + lax.axis_index("c")
        base = wid * b_per_w
        pltpu.sync_copy(idx_hbm.at[wid], idx_v)
        bufs = (buf0, buf1)
        ssems = (ssem0, ssem1)
        for c in range(n_ch):
            buf = bufs[c % 2]
            ssem = ssems[c % 2]
            if c >= 2:
                pltpu.make_async_copy(
                    buf, out_hbm.at[idx_v.at[c - 2]], ssem
                ).wait()
            pltpu.sync_copy(table_hbm.at[pl.ds(base + c * _CH, _CH)], buf)
            pltpu.async_copy(buf, out_hbm.at[idx_v.at[c]], ssem)
        for c in range(max(n_ch - 2, 0), n_ch):
            pltpu.make_async_copy(
                bufs[c % 2], out_hbm.at[idx_v.at[c]], ssems[c % 2]
            ).wait()

    return k


def _mm_body(aux_ref, x_ref, b_ref, w_hbm, y_ref, wb0, wb1, sem0, sem1):
    i = pl.program_id(0)
    te = aux_ref[0, i]
    fetch = aux_ref[1, i]
    slot = aux_ref[2, i]
    nxt = aux_ref[3, i]

    @pl.when(i == 0)
    def _():
        pltpu.make_async_copy(w_hbm.at[te], wb0, sem0).start()

    @pl.when((fetch == 1) & (nxt >= 0) & (slot == 0))
    def _():
        pltpu.make_async_copy(w_hbm.at[nxt], wb1, sem1).start()

    @pl.when((fetch == 1) & (nxt >= 0) & (slot == 1))
    def _():
        pltpu.make_async_copy(w_hbm.at[nxt], wb0, sem0).start()

    @pl.when((fetch == 1) & (slot == 0))
    def _():
        pltpu.make_async_copy(w_hbm.at[te], wb0, sem0).wait()

    @pl.when((fetch == 1) & (slot == 1))
    def _():
        pltpu.make_async_copy(w_hbm.at[te], wb1, sem1).wait()

    x = x_ref[...]

    @pl.when(slot == 0)
    def _():
        y_ref[...] = lax.dot_general(
            x, wb0[...], (((1,), (1,)), ((), ())),
            preferred_element_type=jnp.float32) + b_ref[0]

    @pl.when(slot == 1)
    def _():
        y_ref[...] = lax.dot_general(
            x, wb1[...], (((1,), (1,)), ((), ())),
            preferred_element_type=jnp.float32) + b_ref[0]


def _grouped_matmul(aux, x_padded, W, b):
    P, D = x_padded.shape
    nt = P // T
    grid_spec = pltpu.PrefetchScalarGridSpec(
        num_scalar_prefetch=1,
        grid=(nt,),
        in_specs=[
            pl.BlockSpec((T, D), lambda i, aux: (i, 0)),
            pl.BlockSpec((1, 1, D), lambda i, aux: (aux[0, i], 0, 0)),
            pl.BlockSpec(memory_space=pl.ANY),
        ],
        out_specs=pl.BlockSpec((T, D), lambda i, aux: (i, 0)),
        scratch_shapes=[
            pltpu.VMEM((D, D), jnp.float32),
            pltpu.VMEM((D, D), jnp.float32),
            pltpu.SemaphoreType.DMA,
            pltpu.SemaphoreType.DMA,
        ],
    )
    return pl.pallas_call(
        _mm_body,
        grid_spec=grid_spec,
        out_shape=jax.ShapeDtypeStruct((P, D), jnp.float32),
    )(aux, x_padded, b.reshape(b.shape[0], 1, b.shape[1]), W)


def kernel(xs, mxs, actions, W, b):
    N, D = xs.shape
    E = W.shape[0]
    a = actions.astype(jnp.int32)

    oh = (a[:, None] == jnp.arange(E, dtype=jnp.int32)[None, :]).astype(jnp.int32)
    csum = jnp.cumsum(oh, axis=0)
    hist = csum[-1]
    padded = ((hist + T - 1) // T) * T
    off_end = jnp.cumsum(padded)
    off = off_end - padded
    P = N + E * T
    p = jnp.sum(oh * (off[None, :] + csum - 1), axis=1).astype(jnp.int32)
    nt = P // T
    tile_start = jnp.arange(nt, dtype=jnp.int32)[:, None] * T
    tile_expert = jnp.minimum(
        jnp.sum((tile_start >= off_end[None, :]).astype(jnp.int32), axis=1),
        E - 1)
    change = jnp.concatenate(
        [jnp.ones((1,), jnp.int32),
         (tile_expert[1:] != tile_expert[:-1]).astype(jnp.int32)])
    slot = (jnp.cumsum(change) - 1) % 2
    idxs = jnp.where(change == 1, jnp.arange(nt, dtype=jnp.int32), nt)
    suf_min = lax.associative_scan(jnp.minimum, idxs, reverse=True)
    next_first = jnp.concatenate(
        [suf_min[1:], jnp.full((1,), nt, jnp.int32)])
    next_e = jnp.where(next_first < nt,
                       tile_expert[jnp.minimum(next_first, nt - 1)], -1)
    aux = jnp.stack([tile_expert, change, slot, next_e]).astype(jnp.int32)

    nw = 32
    p3 = p.reshape(nw, -1, _CH)
    x_padded = _make_sc_row_scatter(N, D, P)(xs, p3)

    y_padded = _grouped_matmul(aux, x_padded, W, b)

    ys = _make_sc_row_gather(P, D, N)(y_padded, p)
    return (ys, mxs, actions)


# speedup vs baseline: 1.2287x; 1.0302x over previous
import functools

import jax
import jax.numpy as jnp
from jax import lax
from jax.experimental import pallas as pl
from jax.experimental.pallas import tpu as pltpu
from jax.experimental.pallas import tpu_sc as plsc


T = 256
_CH = 32


@functools.lru_cache(maxsize=None)
def _make_sc_row_gather(R, D, B):
    info = plsc.get_sparse_core_info()
    nw = info.num_cores * info.num_subcores
    b_per_w = B // nw
    assert B % (8 * nw) == 0 and b_per_w % _CH == 0
    n_ch = b_per_w // _CH
    nc = info.num_cores
    mesh = plsc.VectorSubcoreMesh(core_axis_name="c", subcore_axis_name="s")

    @functools.partial(
        pl.kernel,
        out_type=jax.ShapeDtypeStruct((B, D), jnp.float32),
        mesh=mesh,
        scratch_types=[
            pltpu.VMEM((b_per_w,), jnp.int32),
            pltpu.VMEM((_CH, D), jnp.float32),
            pltpu.VMEM((_CH, D), jnp.float32),
            pltpu.VMEM((_CH, D), jnp.float32),
            pltpu.SemaphoreType.DMA,
            pltpu.SemaphoreType.DMA,
            pltpu.SemaphoreType.DMA,
            pltpu.SemaphoreType.DMA,
            pltpu.SemaphoreType.DMA,
            pltpu.SemaphoreType.DMA,
        ],
    )
    def k(table_hbm, idx_hbm, out_hbm, idx_v,
          b0, b1, b2, rs0, rs1, rs2, ss0, ss1, ss2):
        wid = lax.axis_index("s") * nc + lax.axis_index("c")
        base = wid * b_per_w
        pltpu.sync_copy(idx_hbm.at[pl.ds(base, b_per_w)], idx_v)
        bufs = (b0, b1, b2)
        rsems = (rs0, rs1, rs2)
        ssems = (ss0, ss1, ss2)

        def rd(c):
            return pltpu.make_async_copy(
                table_hbm.at[idx_v.at[pl.ds(c * _CH, _CH)]],
                bufs[c % 3], rsems[c % 3])

        def st(c):
            return pltpu.make_async_copy(
                bufs[c % 3], out_hbm.at[pl.ds(base + c * _CH, _CH)],
                ssems[c % 3])

        rd(0).start()
        if n_ch > 1:
            rd(1).start()
        for c in range(n_ch):
            rd(c).wait()
            st(c).start()
            if c + 2 < n_ch:
                if c >= 1:
                    st(c - 1).wait()
                rd(c + 2).start()
        for c in range(max(n_ch - 2, 0), n_ch):
            st(c).wait()

    return k


@functools.lru_cache(maxsize=None)
def _make_sc_row_scatter(B, D, R):
    info = plsc.get_sparse_core_info()
    nw = info.num_cores * info.num_subcores
    b_per_w = B // nw
    assert B % (8 * nw) == 0 and b_per_w % _CH == 0
    n_ch = b_per_w // _CH
    nc = info.num_cores
    mesh = plsc.VectorSubcoreMesh(core_axis_name="c", subcore_axis_name="s")

    @functools.partial(
        pl.kernel,
        out_type=jax.ShapeDtypeStruct((R, D), jnp.float32),
        mesh=mesh,
        scratch_types=[
            pltpu.VMEM((n_ch, _CH), jnp.int32),
            pltpu.VMEM((_CH, D), jnp.float32),
            pltpu.VMEM((_CH, D), jnp.float32),
            pltpu.VMEM((_CH, D), jnp.float32),
            pltpu.SemaphoreType.DMA,
            pltpu.SemaphoreType.DMA,
            pltpu.SemaphoreType.DMA,
            pltpu.SemaphoreType.DMA,
            pltpu.SemaphoreType.DMA,
            pltpu.SemaphoreType.DMA,
        ],
    )
    def k(table_hbm, idx_hbm, out_hbm, idx_v,
          b0, b1, b2, rs0, rs1, rs2, ss0, ss1, ss2):
        wid = lax.axis_index("s") * nc + lax.axis_index("c")
        base = wid * b_per_w
        pltpu.sync_copy(idx_hbm.at[wid], idx_v)
        bufs = (b0, b1, b2)
        rsems = (rs0, rs1, rs2)
        ssems = (ss0, ss1, ss2)

        def rd(c):
            return pltpu.make_async_copy(
                table_hbm.at[pl.ds(base + c * _CH, _CH)],
                bufs[c % 3], rsems[c % 3])

        def st(c):
            return pltpu.make_async_copy(
                bufs[c % 3], out_hbm.at[idx_v.at[c]], ssems[c % 3])

        rd(0).start()
        if n_ch > 1:
            rd(1).start()
        for c in range(n_ch):
            rd(c).wait()
            st(c).start()
            if c + 2 < n_ch:
                if c >= 1:
                    st(c - 1).wait()
                rd(c + 2).start()
        for c in range(max(n_ch - 2, 0), n_ch):
            st(c).wait()

    return k


def _mm_body(aux_ref, x_ref, b_ref, w_hbm, y_ref, wb0, wb1, sem0, sem1):
    i = pl.program_id(0)
    te = aux_ref[0, i]
    fetch = aux_ref[1, i]
    slot = aux_ref[2, i]
    nxt = aux_ref[3, i]

    @pl.when(i == 0)
    def _():
        pltpu.make_async_copy(w_hbm.at[te], wb0, sem0).start()

    @pl.when((fetch == 1) & (nxt >= 0) & (slot == 0))
    def _():
        pltpu.make_async_copy(w_hbm.at[nxt], wb1, sem1).start()

    @pl.when((fetch == 1) & (nxt >= 0) & (slot == 1))
    def _():
        pltpu.make_async_copy(w_hbm.at[nxt], wb0, sem0).start()

    @pl.when((fetch == 1) & (slot == 0))
    def _():
        pltpu.make_async_copy(w_hbm.at[te], wb0, sem0).wait()

    @pl.when((fetch == 1) & (slot == 1))
    def _():
        pltpu.make_async_copy(w_hbm.at[te], wb1, sem1).wait()

    x = x_ref[...]

    @pl.when(slot == 0)
    def _():
        y_ref[...] = lax.dot_general(
            x, wb0[...], (((1,), (1,)), ((), ())),
            preferred_element_type=jnp.float32) + b_ref[0]

    @pl.when(slot == 1)
    def _():
        y_ref[...] = lax.dot_general(
            x, wb1[...], (((1,), (1,)), ((), ())),
            preferred_element_type=jnp.float32) + b_ref[0]


def _grouped_matmul(aux, x_padded, W, b):
    P, D = x_padded.shape
    nt = P // T
    grid_spec = pltpu.PrefetchScalarGridSpec(
        num_scalar_prefetch=1,
        grid=(nt,),
        in_specs=[
            pl.BlockSpec((T, D), lambda i, aux: (i, 0)),
            pl.BlockSpec((1, 1, D), lambda i, aux: (aux[0, i], 0, 0)),
            pl.BlockSpec(memory_space=pl.ANY),
        ],
        out_specs=pl.BlockSpec((T, D), lambda i, aux: (i, 0)),
        scratch_shapes=[
            pltpu.VMEM((D, D), jnp.float32),
            pltpu.VMEM((D, D), jnp.float32),
            pltpu.SemaphoreType.DMA,
            pltpu.SemaphoreType.DMA,
        ],
    )
    return pl.pallas_call(
        _mm_body,
        grid_spec=grid_spec,
        out_shape=jax.ShapeDtypeStruct((P, D), jnp.float32),
    )(aux, x_padded, b.reshape(b.shape[0], 1, b.shape[1]), W)


def kernel(xs, mxs, actions, W, b):
    N, D = xs.shape
    E = W.shape[0]
    a = actions.astype(jnp.int32)

    oh = (a[:, None] == jnp.arange(E, dtype=jnp.int32)[None, :]).astype(jnp.int32)
    csum = jnp.cumsum(oh, axis=0)
    hist = csum[-1]
    padded = ((hist + T - 1) // T) * T
    off_end = jnp.cumsum(padded)
    off = off_end - padded
    P = N + E * T
    p = jnp.sum(oh * (off[None, :] + csum - 1), axis=1).astype(jnp.int32)
    nt = P // T
    tile_start = jnp.arange(nt, dtype=jnp.int32)[:, None] * T
    tile_expert = jnp.minimum(
        jnp.sum((tile_start >= off_end[None, :]).astype(jnp.int32), axis=1),
        E - 1)
    change = jnp.concatenate(
        [jnp.ones((1,), jnp.int32),
         (tile_expert[1:] != tile_expert[:-1]).astype(jnp.int32)])
    slot = (jnp.cumsum(change) - 1) % 2
    idxs = jnp.where(change == 1, jnp.arange(nt, dtype=jnp.int32), nt)
    suf_min = lax.associative_scan(jnp.minimum, idxs, reverse=True)
    next_first = jnp.concatenate(
        [suf_min[1:], jnp.full((1,), nt, jnp.int32)])
    next_e = jnp.where(next_first < nt,
                       tile_expert[jnp.minimum(next_first, nt - 1)], -1)
    aux = jnp.stack([tile_expert, change, slot, next_e]).astype(jnp.int32)

    nw = 32
    p3 = p.reshape(nw, -1, _CH)
    x_padded = _make_sc_row_scatter(N, D, P)(xs, p3)

    y_padded = _grouped_matmul(aux, x_padded, W, b)

    ys = _make_sc_row_gather(P, D, N)(y_padded, p)
    return (ys, mxs, actions)
